# Initial kernel scaffold; baseline (speedup 1.0000x reference)
import jax
import jax.numpy as jnp
from jax.experimental import pallas as pl


def kernel(inputs, weight):
    raise NotImplementedError("write your pallas kernel here")


# speedup vs baseline: 1.3929x; 1.3929x over previous
import functools

import jax
import jax.numpy as jnp
from jax import lax
from jax.experimental import pallas as pl
from jax.experimental.pallas import tpu as pltpu
from jax.experimental.pallas import tpu_sc as plsc

N_EMB = 8192
DIM = 256
N_TOK = 16384
COMMIT = 0.25

BR = 1024
BC = 1024
R = N_TOK // BR
C = N_EMB // BC

NW = 32
PER_W = N_TOK // NW
GCH = 128


def _argmin_body(x_ref, w_ref, idx_ref, sumd_ref, best_val, best_idx):
    c = pl.program_id(1)
    x = x_ref[...]
    w = w_ref[...]
    mm = lax.dot_general(x, w, (((1,), (1,)), ((), ())),
                         preferred_element_type=jnp.float32)
    xsq = jnp.sum(x * x, axis=1, keepdims=True)
    wsq = jnp.sum(w * w, axis=1)[None, :]
    d = (xsq + wsq) - 2.0 * mm
    dmin = jnp.min(d, axis=1, keepdims=True)
    jcol = lax.broadcasted_iota(jnp.int32, (BR, BC), 1)
    lidx = jnp.min(jnp.where(d == dmin, jcol, N_EMB), axis=1, keepdims=True)
    gidx = c * BC + lidx

    @pl.when(c == 0)
    def _():
        best_val[...] = dmin
        best_idx[...] = gidx

    @pl.when(c > 0)
    def _():
        better = dmin < best_val[...]
        best_val[...] = jnp.where(better, dmin, best_val[...])
        best_idx[...] = jnp.where(better, gidx, best_idx[...])

    @pl.when((pl.program_id(0) == 0) & (c == 0))
    def _():
        sumd_ref[...] = jnp.zeros((1, 1), jnp.float32)

    @pl.when(c == C - 1)
    def _():
        idx_ref[...] = best_idx[...]
        sumd_ref[...] += jnp.sum(best_val[...]).reshape(1, 1)


_argmin_call = pl.pallas_call(
    _argmin_body,
    grid=(R, C),
    in_specs=[
        pl.BlockSpec((BR, DIM), lambda r, c: (r, 0)),
        pl.BlockSpec((BC, DIM), lambda r, c: (c, 0)),
    ],
    out_specs=[
        pl.BlockSpec((BR, 1), lambda r, c: (r, 0)),
        pl.BlockSpec((1, 1), lambda r, c: (0, 0)),
    ],
    out_shape=[
        jax.ShapeDtypeStruct((N_TOK, 1), jnp.int32),
        jax.ShapeDtypeStruct((1, 1), jnp.float32),
    ],
    scratch_shapes=[
        pltpu.VMEM((BR, 1), jnp.float32),
        pltpu.VMEM((BR, 1), jnp.int32),
    ],
    compiler_params=pltpu.CompilerParams(
        dimension_semantics=("arbitrary", "arbitrary")),
)


@functools.cache
def _sc_gather_hist_call():
    @functools.partial(
        pl.kernel,
        mesh=plsc.VectorSubcoreMesh(core_axis_name="c", subcore_axis_name="s"),
        out_type=[
            jax.ShapeDtypeStruct((N_TOK, DIM), jnp.float32),
            jax.ShapeDtypeStruct((NW, N_EMB), jnp.int32),
        ],
        scratch_types=[
            pltpu.VMEM((PER_W,), jnp.int32),
            pltpu.VMEM((PER_W // GCH, GCH), jnp.int32),
            pltpu.VMEM((GCH, DIM), jnp.float32),
            pltpu.VMEM((N_EMB,), jnp.int32),
            pltpu.SemaphoreType.DMA,
        ],
        compiler_params=pltpu.CompilerParams(needs_layout_passes=False),
    )
    def _sc_gather_hist(w_hbm, idx_hbm, out_hbm, hist_hbm,
                        idx_flat, idx_2d, rows_v, hist_v, sem):
        wid = lax.axis_index("s") * 2 + lax.axis_index("c")
        base = wid * PER_W
        pltpu.sync_copy(idx_hbm.at[pl.ds(base, PER_W)], idx_flat)

        def _zero(i, carry):
            hist_v[pl.ds(i * 16, 16)] = jnp.zeros((16,), jnp.int32)
            return carry
        lax.fori_loop(0, N_EMB // 16, _zero, 0)

        ones = jnp.ones((16,), jnp.int32)

        def _hist(i, carry):
            ii = idx_flat[pl.ds(i * 16, 16)]
            plsc.addupdate_scatter(hist_v, [ii], ones)
            return carry
        lax.fori_loop(0, PER_W // 16, _hist, 0)
        pltpu.sync_copy(hist_v, hist_hbm.at[wid])

        for ch in range(PER_W // GCH):
            pltpu.sync_copy(idx_hbm.at[pl.ds(base + ch * GCH, GCH)],
                            idx_2d.at[ch])
            pltpu.async_copy(w_hbm.at[idx_2d.at[ch]], rows_v, sem).wait()
            pltpu.sync_copy(rows_v, out_hbm.at[pl.ds(base + ch * GCH, GCH)])

    return _sc_gather_hist


def _fin_body(sumd_ref, hist_ref, loss_ref, perp_ref):
    counts = jnp.sum(hist_ref[...].astype(jnp.float32), axis=0, keepdims=True)
    p = counts * (1.0 / N_TOK)
    ent = jnp.sum(p * jnp.log(p + 1e-10))
    loss_ref[...] = sumd_ref[...] * ((1.0 + COMMIT) / (N_TOK * DIM))
    perp_ref[...] = jnp.exp(-ent).reshape(1, 1)


_fin_call = pl.pallas_call(
    _fin_body,
    out_shape=[
        jax.ShapeDtypeStruct((1, 1), jnp.float32),
        jax.ShapeDtypeStruct((1, 1), jnp.float32),
    ],
)


def kernel(inputs, weight):
    flat = inputs.reshape(-1, DIM)
    idx2d, sumd = _argmin_call(flat, weight)
    idx = idx2d[:, 0]
    quant, hists = _sc_gather_hist_call()(weight, idx)
    loss2, perp2 = _fin_call(sumd, hists)
    return quant.reshape(inputs.shape), loss2[0, 0], perp2[0, 0]
